# baseline (speedup 1.0000x reference)
import functools

import jax
import jax.numpy as jnp
from jax import lax
from jax.experimental import pallas as pl
from jax.experimental.pallas import tpu as pltpu, tpu_sc as plsc

TEMPERATURE = 0.1
EPS = 1e-07
N = 4096
NC, NS, L = 2, 16, 16
NW = NC * NS
ROWS_PER_W = N // NW
RBLK = 16
NBLK = ROWS_PER_W // RBLK
NCH = N // L
_LN2 = 0.6931471805599453


def _sims_body(e_rows_ref, e_full_ref, out_ref, aux_ref):
    ef = e_full_ref[...]
    nf = ef / jnp.maximum(jnp.sqrt(jnp.sum(ef * ef, axis=1, keepdims=True)), 1e-12)
    er = e_rows_ref[...]
    nr = er / jnp.maximum(jnp.sqrt(jnp.sum(er * er, axis=1, keepdims=True)), 1e-12)
    s = jnp.dot(nr, nf.T, preferred_element_type=jnp.float32) * (1.0 / TEMPERATURE)
    out_ref[...] = s
    bsum = jnp.sum(s)
    bdiag = jnp.sum(nr * nr) * (1.0 / TEMPERATURE)
    aux_ref[...] = jnp.full((1, 1, 128), (bsum - bdiag) * (1.0 / 128.0), jnp.float32)


def _sims(embeddings):
    n, d = embeddings.shape
    br = 512
    g = n // br
    return pl.pallas_call(
        _sims_body,
        grid=(g,),
        in_specs=[
            pl.BlockSpec((br, d), lambda i: (i, 0)),
            pl.BlockSpec((n, d), lambda i: (0, 0)),
        ],
        out_specs=[
            pl.BlockSpec((br, n), lambda i: (i, 0)),
            pl.BlockSpec((1, 1, 128), lambda i: (i, 0, 0)),
        ],
        out_shape=[
            jax.ShapeDtypeStruct((n, n), jnp.float32),
            jax.ShapeDtypeStruct((g, 1, 128), jnp.float32),
        ],
    )(embeddings, embeddings)


def _log_f32(x):
    bits = lax.bitcast_convert_type(x, jnp.int32)
    ex = (lax.shift_right_logical(bits, 23) & 255) - 127
    man = lax.bitcast_convert_type((bits & 0x007FFFFF) | 0x3F800000, jnp.float32)
    r = (man - 1.0) / (man + 1.0)
    r2 = r * r
    p = 2.0 / 9.0
    p = p * r2 + 2.0 / 7.0
    p = p * r2 + 2.0 / 5.0
    p = p * r2 + 2.0 / 3.0
    p = p * r2 + 2.0
    return ex.astype(jnp.float32) * _LN2 + r * p


def _lane_bcast(vec, idxvec):
    return lax.gather(
        vec, idxvec[:, None],
        dimension_numbers=lax.GatherDimensionNumbers(
            offset_dims=(), collapsed_slice_dims=(0,), start_index_map=(0,)),
        slice_sizes=(1,),
        mode=lax.GatherScatterMode.PROMISE_IN_BOUNDS)


def _sortable_key(bits):
    return jnp.where(bits >= 0, bits, jnp.int32(-2147483648) - bits)


def _sc_body(s0_hbm, s1_hbm, ys_hbm, rl_hbm, rr_hbm, part_hbm,
             ys_v, ysk_v, rl_v, rr_v, rows_v, e_v, q_v, r_v,
             acc_v, sem):
    wid = lax.axis_index("s") * NC + lax.axis_index("c")
    lane = lax.iota(jnp.int32, L)

    for d, sd_hbm in enumerate((s0_hbm, s1_hbm)):
        pltpu.sync_copy(ys_hbm.at[d], ys_v)
        pltpu.sync_copy(rl_hbm.at[d], rl_v)
        pltpu.sync_copy(rr_hbm.at[d], rr_v)

        @plsc.parallel_loop(0, NCH + 1, unroll=4)
        def _build(c):
            off = c * L
            src = jnp.clip(lane + (off - 1), 0, N - 1)
            b = lax.bitcast_convert_type(plsc.load_gather(ys_v, [src]), jnp.int32)
            ysk_v[pl.ds(off, L)] = _sortable_key(b)

        r_v[pl.ds(N, L)] = jnp.zeros((L,), jnp.float32)

        kmid = plsc.load_gather(ysk_v, [jnp.full((L,), 2048, jnp.int32)])
        k1q = plsc.load_gather(ysk_v, [jnp.full((L,), 1024, jnp.int32)])
        k3q = plsc.load_gather(ysk_v, [jnp.full((L,), 3072, jnp.int32)])
        ktop = plsc.load_gather(ysk_v, [jnp.full((L,), N, jnp.int32)])

        def blk_body(b, acc, sd_hbm=sd_hbm):
            base = wid * ROWS_PER_W + b * RBLK
            pltpu.async_copy(sd_hbm.at[pl.ds(base, RBLK)], rows_v, sem).wait()

            def row_body(r, acc):
                isr = base + r
                yi = plsc.load_gather(ys_v, [jnp.full((L,), isr, jnp.int32)])
                yi2 = yi + yi

                lastl = jnp.full((L,), L - 1, jnp.int32)

                @plsc.parallel_loop(0, NCH, unroll=8,
                                    carry=jnp.zeros((L,), jnp.float32))
                def p1(c, carry):
                    off = c * L
                    e = jnp.exp(rows_v[r, pl.ds(off, L)])
                    e_v[pl.ds(off, L)] = e
                    cs = plsc.cumsum(e)
                    q_v[pl.ds(off, L)] = cs + carry - e
                    return carry + _lane_bcast(cs, lastl)

                q_v[pl.ds(N, L)] = p1

                @plsc.parallel_loop(0, NCH, unroll=8,
                                    carry=jnp.zeros((L,), jnp.float32))
                def p2(c2, carry):
                    off = (NCH - 1 - c2) * L
                    e = e_v[pl.ds(off, L)]
                    cs = plsc.cumsum(e)
                    tot = _lane_bcast(cs, lastl)
                    r_v[pl.ds(off, L)] = carry + tot - cs + e
                    return carry + tot

                mn_carry = (jnp.ones((L,), jnp.float32),
                            jnp.zeros((L,), jnp.int32))

                @plsc.parallel_loop(0, NCH, unroll=16, carry=mn_carry)
                def mn(c, carry):
                    prodm, eacc = carry
                    off = c * L
                    yj = ys_v[pl.ds(off, L)]
                    rlc = rl_v[pl.ds(off, L)]
                    rrc = rr_v[pl.ds(off, L)]
                    ge = yj >= yi
                    m = yi2 - yj
                    mk = _sortable_key(lax.bitcast_convert_type(m, jnp.int32))
                    mk = mk + (yj > yi).astype(jnp.int32)
                    ok1 = kmid < mk
                    cnt = jnp.where(ok1, 2048, 0)
                    t2 = jnp.where(ok1, k3q, k1q)
                    cnt = jnp.where(t2 < mk, cnt + 1024, cnt)
                    for bit in (512, 256, 128, 64, 32, 16, 8, 4, 2, 1):
                        cand = cnt + bit
                        t = plsc.load_gather(ysk_v, [cand])
                        cnt = jnp.where(t < mk, cand, cnt)
                    cnt = jnp.where(ktop < mk, N, cnt)
                    lo = jnp.where(ge, cnt, rrc)
                    hi = jnp.where(ge, rlc, cnt)
                    qv = plsc.load_gather(q_v, [lo])
                    rv = plsc.load_gather(r_v, [hi])
                    dbits = lax.bitcast_convert_type(
                        jnp.maximum(qv + rv, EPS), jnp.int32)
                    eacc = eacc + (lax.shift_right_logical(dbits, 23) & 255)
                    man = lax.bitcast_convert_type(
                        (dbits & 0x007FFFFF) | 0x3F800000, jnp.float32)
                    t = prodm * man
                    big = t >= 2.0
                    prodm = jnp.where(big, t * 0.5, t)
                    eacc = eacc + big.astype(jnp.int32)
                    return prodm, eacc

                prodm, eacc = mn
                stot = plsc.load_gather(q_v, [jnp.full((L,), N, jnp.int32)])
                dterm = _log_f32(jnp.maximum(stot, EPS))
                row_log = (_log_f32(prodm)
                           + (eacc.astype(jnp.float32) - 127.0 * NCH) * _LN2)
                return acc + row_log - jnp.where(lane == 0, dterm, 0.0)

            return lax.fori_loop(0, RBLK, row_body, acc)

        acc = lax.fori_loop(0, NBLK, blk_body, jnp.zeros((L,), jnp.float32))
        acc_v[...] = acc
        pltpu.sync_copy(acc_v, part_hbm.at[d, wid])


@functools.partial(
    pl.kernel,
    mesh=plsc.VectorSubcoreMesh(core_axis_name="c", subcore_axis_name="s"),
    out_type=jax.ShapeDtypeStruct((2, NW, L), jnp.float32),
    compiler_params=pltpu.CompilerParams(needs_layout_passes=False),
    scratch_types=[
        pltpu.VMEM((N,), jnp.float32),
        pltpu.VMEM((N + L,), jnp.int32),
        pltpu.VMEM((N,), jnp.int32),
        pltpu.VMEM((N,), jnp.int32),
        pltpu.VMEM((RBLK, N), jnp.float32),
        pltpu.VMEM((N,), jnp.float32),
        pltpu.VMEM((N + L,), jnp.float32),
        pltpu.VMEM((N + L,), jnp.float32),
        pltpu.VMEM((L,), jnp.float32),
        pltpu.SemaphoreType.DMA,
    ],
)
def _sc_loss(s0_hbm, s1_hbm, ys_hbm, rl_hbm, rr_hbm, part_hbm, *scratch):
    _sc_body(s0_hbm, s1_hbm, ys_hbm, rl_hbm, rr_hbm, part_hbm, *scratch)


def kernel(embeddings, labels):
    n, _ = embeddings.shape
    assert n == N and labels.shape == (N, 2)

    iota = lax.iota(jnp.int32, N)
    sims_l, ys_l, rl_l, rr_l = [], [], [], []
    aux = None
    for d in range(2):
        y = labels[:, d]
        ys, order = lax.sort_key_val(y, iota)
        ys_l.append(ys)
        sims_d, aux_d = _sims(embeddings[order])
        sims_l.append(sims_d)
        if aux is None:
            aux = aux_d
        neq_prev = jnp.concatenate([jnp.ones((1,), jnp.bool_), ys[1:] != ys[:-1]])
        rl_l.append(lax.cummax(jnp.where(neq_prev, iota, 0)))
        neq_next = jnp.concatenate([ys[1:] != ys[:-1], jnp.ones((1,), jnp.bool_)])
        rr_l.append(N - jnp.flip(lax.cummax(jnp.where(jnp.flip(neq_next), iota, 0))))

    offdiag_sims = jnp.sum(aux)
    part = _sc_loss(
        sims_l[0],
        sims_l[1],
        jnp.stack(ys_l),
        jnp.stack(rl_l),
        jnp.stack(rr_l),
    )
    log_sums = jnp.sum(part, axis=(1, 2))
    return (log_sums - offdiag_sims) / (N * (N - 1))


# speedup vs baseline: 1.0495x; 1.0495x over previous
import functools

import jax
import jax.numpy as jnp
from jax import lax
from jax.experimental import pallas as pl
from jax.experimental.pallas import tpu as pltpu, tpu_sc as plsc

TEMPERATURE = 0.1
EPS = 1e-07
N = 4096
NC, NS, L = 2, 16, 16
NW = NC * NS
ROWS_PER_W = N // NW
RBLK = 16
NBLK = ROWS_PER_W // RBLK
NCH = N // L
_LN2 = 0.6931471805599453


def _sims_body(e_rows_ref, e_full_ref, out_ref, aux_ref):
    ef = e_full_ref[...]
    nf = ef / jnp.maximum(jnp.sqrt(jnp.sum(ef * ef, axis=1, keepdims=True)), 1e-12)
    er = e_rows_ref[...]
    nr = er / jnp.maximum(jnp.sqrt(jnp.sum(er * er, axis=1, keepdims=True)), 1e-12)
    s = jnp.dot(nr, nf.T, preferred_element_type=jnp.float32) * (1.0 / TEMPERATURE)
    out_ref[...] = s
    bsum = jnp.sum(s)
    bdiag = jnp.sum(nr * nr) * (1.0 / TEMPERATURE)
    aux_ref[...] = jnp.full((1, 1, 128), (bsum - bdiag) * (1.0 / 128.0), jnp.float32)


def _sims(embeddings):
    n, d = embeddings.shape
    br = 512
    g = n // br
    return pl.pallas_call(
        _sims_body,
        grid=(g,),
        in_specs=[
            pl.BlockSpec((br, d), lambda i: (i, 0)),
            pl.BlockSpec((n, d), lambda i: (0, 0)),
        ],
        out_specs=[
            pl.BlockSpec((br, n), lambda i: (i, 0)),
            pl.BlockSpec((1, 1, 128), lambda i: (i, 0, 0)),
        ],
        out_shape=[
            jax.ShapeDtypeStruct((n, n), jnp.float32),
            jax.ShapeDtypeStruct((g, 1, 128), jnp.float32),
        ],
    )(embeddings, embeddings)


def _log_f32(x):
    bits = lax.bitcast_convert_type(x, jnp.int32)
    ex = (lax.shift_right_logical(bits, 23) & 255) - 127
    man = lax.bitcast_convert_type((bits & 0x007FFFFF) | 0x3F800000, jnp.float32)
    r = (man - 1.0) / (man + 1.0)
    r2 = r * r
    p = 2.0 / 9.0
    p = p * r2 + 2.0 / 7.0
    p = p * r2 + 2.0 / 5.0
    p = p * r2 + 2.0 / 3.0
    p = p * r2 + 2.0
    return ex.astype(jnp.float32) * _LN2 + r * p


def _lane_bcast(vec, idxvec):
    return lax.gather(
        vec, idxvec[:, None],
        dimension_numbers=lax.GatherDimensionNumbers(
            offset_dims=(), collapsed_slice_dims=(0,), start_index_map=(0,)),
        slice_sizes=(1,),
        mode=lax.GatherScatterMode.PROMISE_IN_BOUNDS)


def _sortable_key(bits):
    return jnp.where(bits >= 0, bits, jnp.int32(-2147483648) - bits)


def _sc_body(s0_hbm, s1_hbm, ys_hbm, rl_hbm, rr_hbm, part_hbm,
             ys_v, ysk_v, rl_v, rr_v, rows_v, e_v, q_v, r_v,
             acc_v, sem):
    wid = lax.axis_index("s") * NC + lax.axis_index("c")
    lane = lax.iota(jnp.int32, L)

    for d, sd_hbm in enumerate((s0_hbm, s1_hbm)):
        pltpu.sync_copy(ys_hbm.at[d], ys_v)
        pltpu.sync_copy(rl_hbm.at[d], rl_v)
        pltpu.sync_copy(rr_hbm.at[d], rr_v)

        @plsc.parallel_loop(0, NCH + 1, unroll=4)
        def _build(c):
            off = c * L
            src = jnp.clip(lane + (off - 1), 0, N - 1)
            b = lax.bitcast_convert_type(plsc.load_gather(ys_v, [src]), jnp.int32)
            ysk_v[pl.ds(off, L)] = _sortable_key(b)

        r_v[pl.ds(N, L)] = jnp.zeros((L,), jnp.float32)

        kmid = plsc.load_gather(ysk_v, [jnp.full((L,), 2048, jnp.int32)])
        k1q = plsc.load_gather(ysk_v, [jnp.full((L,), 1024, jnp.int32)])
        k3q = plsc.load_gather(ysk_v, [jnp.full((L,), 3072, jnp.int32)])
        ktop = plsc.load_gather(ysk_v, [jnp.full((L,), N, jnp.int32)])

        def blk_body(b, acc, sd_hbm=sd_hbm):
            base = wid * ROWS_PER_W + b * RBLK
            pltpu.async_copy(sd_hbm.at[pl.ds(base, RBLK)], rows_v, sem).wait()

            def row_body(r, acc):
                isr = base + r
                yi = plsc.load_gather(ys_v, [jnp.full((L,), isr, jnp.int32)])
                yi2 = yi + yi

                lastl = jnp.full((L,), L - 1, jnp.int32)

                @plsc.parallel_loop(0, NCH, unroll=8,
                                    carry=jnp.zeros((L,), jnp.float32))
                def p1(c, carry):
                    off = c * L
                    e = jnp.exp(rows_v[r, pl.ds(off, L)])
                    e_v[pl.ds(off, L)] = e
                    cs = plsc.cumsum(e)
                    q_v[pl.ds(off, L)] = cs + carry - e
                    return carry + _lane_bcast(cs, lastl)

                q_v[pl.ds(N, L)] = p1

                @plsc.parallel_loop(0, NCH, unroll=8,
                                    carry=jnp.zeros((L,), jnp.float32))
                def p2(c2, carry):
                    off = (NCH - 1 - c2) * L
                    e = e_v[pl.ds(off, L)]
                    cs = plsc.cumsum(e)
                    tot = _lane_bcast(cs, lastl)
                    r_v[pl.ds(off, L)] = carry + tot - cs + e
                    return carry + tot

                mn_carry = (jnp.ones((L,), jnp.float32),
                            jnp.zeros((L,), jnp.int32))

                @plsc.parallel_loop(0, NCH, unroll=8, carry=mn_carry)
                def mn(c, carry):
                    prodm, eacc = carry
                    off = c * L
                    yj = ys_v[pl.ds(off, L)]
                    rlc = rl_v[pl.ds(off, L)]
                    rrc = rr_v[pl.ds(off, L)]
                    ge = yj >= yi
                    m = yi2 - yj
                    mk = _sortable_key(lax.bitcast_convert_type(m, jnp.int32))
                    mk = mk + (yj > yi).astype(jnp.int32)
                    ok1 = kmid < mk
                    cnt = jnp.where(ok1, 2048, 0)
                    t2 = jnp.where(ok1, k3q, k1q)
                    cnt = jnp.where(t2 < mk, cnt + 1024, cnt)
                    for bit in (512, 256, 128, 64, 32, 16, 8, 4, 2, 1):
                        cand = cnt + bit
                        t = plsc.load_gather(ysk_v, [cand])
                        cnt = jnp.where(t < mk, cand, cnt)
                    cnt = jnp.where(ktop < mk, N, cnt)
                    lo = jnp.where(ge, cnt, rrc)
                    hi = jnp.where(ge, rlc, cnt)
                    qv = plsc.load_gather(q_v, [lo])
                    rv = plsc.load_gather(r_v, [hi])
                    dbits = lax.bitcast_convert_type(
                        jnp.maximum(qv + rv, EPS), jnp.int32)
                    eacc = eacc + (lax.shift_right_logical(dbits, 23) & 255)
                    man = lax.bitcast_convert_type(
                        (dbits & 0x007FFFFF) | 0x3F800000, jnp.float32)
                    t = prodm * man
                    big = t >= 2.0
                    prodm = jnp.where(big, t * 0.5, t)
                    eacc = eacc + big.astype(jnp.int32)
                    return prodm, eacc

                prodm, eacc = mn
                stot = plsc.load_gather(q_v, [jnp.full((L,), N, jnp.int32)])
                dterm = _log_f32(jnp.maximum(stot, EPS))
                row_log = (_log_f32(prodm)
                           + (eacc.astype(jnp.float32) - 127.0 * NCH) * _LN2)
                return acc + row_log - jnp.where(lane == 0, dterm, 0.0)

            return lax.fori_loop(0, RBLK, row_body, acc)

        acc = lax.fori_loop(0, NBLK, blk_body, jnp.zeros((L,), jnp.float32))
        acc_v[...] = acc
        pltpu.sync_copy(acc_v, part_hbm.at[d, wid])


@functools.partial(
    pl.kernel,
    mesh=plsc.VectorSubcoreMesh(core_axis_name="c", subcore_axis_name="s"),
    out_type=jax.ShapeDtypeStruct((2, NW, L), jnp.float32),
    compiler_params=pltpu.CompilerParams(needs_layout_passes=False),
    scratch_types=[
        pltpu.VMEM((N,), jnp.float32),
        pltpu.VMEM((N + L,), jnp.int32),
        pltpu.VMEM((N,), jnp.int32),
        pltpu.VMEM((N,), jnp.int32),
        pltpu.VMEM((RBLK, N), jnp.float32),
        pltpu.VMEM((N,), jnp.float32),
        pltpu.VMEM((N + L,), jnp.float32),
        pltpu.VMEM((N + L,), jnp.float32),
        pltpu.VMEM((L,), jnp.float32),
        pltpu.SemaphoreType.DMA,
    ],
)
def _sc_loss(s0_hbm, s1_hbm, ys_hbm, rl_hbm, rr_hbm, part_hbm, *scratch):
    _sc_body(s0_hbm, s1_hbm, ys_hbm, rl_hbm, rr_hbm, part_hbm, *scratch)


def kernel(embeddings, labels):
    n, _ = embeddings.shape
    assert n == N and labels.shape == (N, 2)

    iota = lax.iota(jnp.int32, N)
    sims_l, ys_l, rl_l, rr_l = [], [], [], []
    aux = None
    for d in range(2):
        y = labels[:, d]
        ys, order = lax.sort_key_val(y, iota)
        ys_l.append(ys)
        sims_d, aux_d = _sims(embeddings[order])
        sims_l.append(sims_d)
        if aux is None:
            aux = aux_d
        neq_prev = jnp.concatenate([jnp.ones((1,), jnp.bool_), ys[1:] != ys[:-1]])
        rl_l.append(lax.cummax(jnp.where(neq_prev, iota, 0)))
        neq_next = jnp.concatenate([ys[1:] != ys[:-1], jnp.ones((1,), jnp.bool_)])
        rr_l.append(N - jnp.flip(lax.cummax(jnp.where(jnp.flip(neq_next), iota, 0))))

    offdiag_sims = jnp.sum(aux)
    part = _sc_loss(
        sims_l[0],
        sims_l[1],
        jnp.stack(ys_l),
        jnp.stack(rl_l),
        jnp.stack(rr_l),
    )
    log_sums = jnp.sum(part, axis=(1, 2))
    return (log_sums - offdiag_sims) / (N * (N - 1))


# speedup vs baseline: 1.0749x; 1.0241x over previous
import functools

import jax
import jax.numpy as jnp
from jax import lax
from jax.experimental import pallas as pl
from jax.experimental.pallas import tpu as pltpu, tpu_sc as plsc

TEMPERATURE = 0.1
EPS = 1e-07
N = 4096
NC, NS, L = 2, 16, 16
NW = NC * NS
ROWS_PER_W = N // NW
RBLK = 8
NBLK = ROWS_PER_W // RBLK
NCH = N // L
_LN2 = 0.6931471805599453


def _sims_body(e_rows_ref, e_full_ref, out_ref, aux_ref):
    ef = e_full_ref[...]
    nf = ef / jnp.maximum(jnp.sqrt(jnp.sum(ef * ef, axis=1, keepdims=True)), 1e-12)
    er = e_rows_ref[...]
    nr = er / jnp.maximum(jnp.sqrt(jnp.sum(er * er, axis=1, keepdims=True)), 1e-12)
    s = jnp.dot(nr, nf.T, preferred_element_type=jnp.float32) * (1.0 / TEMPERATURE)
    out_ref[...] = s
    bsum = jnp.sum(s)
    bdiag = jnp.sum(nr * nr) * (1.0 / TEMPERATURE)
    aux_ref[...] = jnp.full((1, 1, 128), (bsum - bdiag) * (1.0 / 128.0), jnp.float32)


def _sims(embeddings):
    n, d = embeddings.shape
    br = 512
    g = n // br
    return pl.pallas_call(
        _sims_body,
        grid=(g,),
        in_specs=[
            pl.BlockSpec((br, d), lambda i: (i, 0)),
            pl.BlockSpec((n, d), lambda i: (0, 0)),
        ],
        out_specs=[
            pl.BlockSpec((br, n), lambda i: (i, 0)),
            pl.BlockSpec((1, 1, 128), lambda i: (i, 0, 0)),
        ],
        out_shape=[
            jax.ShapeDtypeStruct((n, n), jnp.float32),
            jax.ShapeDtypeStruct((g, 1, 128), jnp.float32),
        ],
    )(embeddings, embeddings)


def _log_f32(x):
    bits = lax.bitcast_convert_type(x, jnp.int32)
    ex = (lax.shift_right_logical(bits, 23) & 255) - 127
    man = lax.bitcast_convert_type((bits & 0x007FFFFF) | 0x3F800000, jnp.float32)
    r = (man - 1.0) / (man + 1.0)
    r2 = r * r
    p = 2.0 / 9.0
    p = p * r2 + 2.0 / 7.0
    p = p * r2 + 2.0 / 5.0
    p = p * r2 + 2.0 / 3.0
    p = p * r2 + 2.0
    return ex.astype(jnp.float32) * _LN2 + r * p


def _lane_bcast(vec, idxvec):
    return lax.gather(
        vec, idxvec[:, None],
        dimension_numbers=lax.GatherDimensionNumbers(
            offset_dims=(), collapsed_slice_dims=(0,), start_index_map=(0,)),
        slice_sizes=(1,),
        mode=lax.GatherScatterMode.PROMISE_IN_BOUNDS)


def _sortable_key(bits):
    return jnp.where(bits >= 0, bits, jnp.int32(-2147483648) - bits)


def _sc_body(s0_hbm, s1_hbm, ys_hbm, rl_hbm, rr_hbm, part_hbm,
             ys_v, ysk_v, rl_v, rr_v, rows_a, rows_b, e_v, q_v, r_v,
             acc_v, sema, semb):
    wid = lax.axis_index("s") * NC + lax.axis_index("c")
    lane = lax.iota(jnp.int32, L)

    for d, sd_hbm in enumerate((s0_hbm, s1_hbm)):
        pltpu.sync_copy(ys_hbm.at[d], ys_v)
        pltpu.sync_copy(rl_hbm.at[d], rl_v)
        pltpu.sync_copy(rr_hbm.at[d], rr_v)

        @plsc.parallel_loop(0, NCH + 1, unroll=4)
        def _build(c):
            off = c * L
            src = jnp.clip(lane + (off - 1), 0, N - 1)
            b = lax.bitcast_convert_type(plsc.load_gather(ys_v, [src]), jnp.int32)
            ysk_v[pl.ds(off, L)] = _sortable_key(b)

        r_v[pl.ds(N, L)] = jnp.zeros((L,), jnp.float32)

        kmid = plsc.load_gather(ysk_v, [jnp.full((L,), 2048, jnp.int32)])
        k1q = plsc.load_gather(ysk_v, [jnp.full((L,), 1024, jnp.int32)])
        k3q = plsc.load_gather(ysk_v, [jnp.full((L,), 3072, jnp.int32)])
        ktop = plsc.load_gather(ysk_v, [jnp.full((L,), N, jnp.int32)])

        wbase = wid * ROWS_PER_W

        def process_block(rows_v, base, acc):
            def row_body(r, acc):
                isr = base + r
                yi = plsc.load_gather(ys_v, [jnp.full((L,), isr, jnp.int32)])
                yi2 = yi + yi

                lastl = jnp.full((L,), L - 1, jnp.int32)

                @plsc.parallel_loop(0, NCH, unroll=8,
                                    carry=jnp.zeros((L,), jnp.float32))
                def p1(c, carry):
                    off = c * L
                    e = jnp.exp(rows_v[r, pl.ds(off, L)])
                    e_v[pl.ds(off, L)] = e
                    cs = plsc.cumsum(e)
                    q_v[pl.ds(off, L)] = cs + carry - e
                    return carry + _lane_bcast(cs, lastl)

                q_v[pl.ds(N, L)] = p1

                @plsc.parallel_loop(0, NCH, unroll=8,
                                    carry=jnp.zeros((L,), jnp.float32))
                def p2(c2, carry):
                    off = (NCH - 1 - c2) * L
                    e = e_v[pl.ds(off, L)]
                    cs = plsc.cumsum(e)
                    tot = _lane_bcast(cs, lastl)
                    r_v[pl.ds(off, L)] = carry + tot - cs + e
                    return carry + tot

                mn_carry = (jnp.ones((L,), jnp.float32),
                            jnp.zeros((L,), jnp.int32))

                @plsc.parallel_loop(0, NCH, unroll=8, carry=mn_carry)
                def mn(c, carry):
                    prodm, eacc = carry
                    off = c * L
                    yj = ys_v[pl.ds(off, L)]
                    rlc = rl_v[pl.ds(off, L)]
                    rrc = rr_v[pl.ds(off, L)]
                    ge = yj >= yi
                    m = yi2 - yj
                    mk = _sortable_key(lax.bitcast_convert_type(m, jnp.int32))
                    mk = mk + (yj > yi).astype(jnp.int32)
                    ok1 = kmid < mk
                    cnt = jnp.where(ok1, 2048, 0)
                    t2 = jnp.where(ok1, k3q, k1q)
                    cnt = jnp.where(t2 < mk, cnt + 1024, cnt)
                    for bit in (512, 256, 128, 64, 32, 16, 8, 4, 2, 1):
                        cand = cnt + bit
                        t = plsc.load_gather(ysk_v, [cand])
                        cnt = jnp.where(t < mk, cand, cnt)
                    cnt = jnp.where(ktop < mk, N, cnt)
                    lo = jnp.where(ge, cnt, rrc)
                    hi = jnp.where(ge, rlc, cnt)
                    qv = plsc.load_gather(q_v, [lo])
                    rv = plsc.load_gather(r_v, [hi])
                    dbits = lax.bitcast_convert_type(
                        jnp.maximum(qv + rv, EPS), jnp.int32)
                    eacc = eacc + (lax.shift_right_logical(dbits, 23) & 255)
                    man = lax.bitcast_convert_type(
                        (dbits & 0x007FFFFF) | 0x3F800000, jnp.float32)
                    t = prodm * man
                    big = t >= 2.0
                    prodm = jnp.where(big, t * 0.5, t)
                    eacc = eacc + big.astype(jnp.int32)
                    return prodm, eacc

                prodm, eacc = mn
                stot = plsc.load_gather(q_v, [jnp.full((L,), N, jnp.int32)])
                dterm = _log_f32(jnp.maximum(stot, EPS))
                row_log = (_log_f32(prodm)
                           + (eacc.astype(jnp.float32) - 127.0 * NCH) * _LN2)
                return acc + row_log - jnp.where(lane == 0, dterm, 0.0)

            return lax.fori_loop(0, RBLK, row_body, acc)

        pltpu.async_copy(sd_hbm.at[pl.ds(wbase, RBLK)], rows_a, sema)
        last_base = wbase + (NBLK - 1) * RBLK

        def outer(t, acc, sd_hbm=sd_hbm):
            base0 = wbase + t * (2 * RBLK)
            pltpu.async_copy(sd_hbm.at[pl.ds(base0 + RBLK, RBLK)], rows_b, semb)
            pltpu.make_async_copy(
                sd_hbm.at[pl.ds(wbase, RBLK)], rows_a, sema).wait()
            acc = process_block(rows_a, base0, acc)
            nxt = jnp.minimum(base0 + 2 * RBLK, last_base)
            pltpu.async_copy(sd_hbm.at[pl.ds(nxt, RBLK)], rows_a, sema)
            pltpu.make_async_copy(
                sd_hbm.at[pl.ds(wbase, RBLK)], rows_b, semb).wait()
            return process_block(rows_b, base0 + RBLK, acc)

        acc = lax.fori_loop(0, NBLK // 2, outer, jnp.zeros((L,), jnp.float32))
        pltpu.make_async_copy(sd_hbm.at[pl.ds(wbase, RBLK)], rows_a, sema).wait()
        acc_v[...] = acc
        pltpu.sync_copy(acc_v, part_hbm.at[d, wid])


@functools.partial(
    pl.kernel,
    mesh=plsc.VectorSubcoreMesh(core_axis_name="c", subcore_axis_name="s"),
    out_type=jax.ShapeDtypeStruct((2, NW, L), jnp.float32),
    compiler_params=pltpu.CompilerParams(needs_layout_passes=False),
    scratch_types=[
        pltpu.VMEM((N,), jnp.float32),
        pltpu.VMEM((N + L,), jnp.int32),
        pltpu.VMEM((N,), jnp.int32),
        pltpu.VMEM((N,), jnp.int32),
        pltpu.VMEM((RBLK, N), jnp.float32),
        pltpu.VMEM((RBLK, N), jnp.float32),
        pltpu.VMEM((N,), jnp.float32),
        pltpu.VMEM((N + L,), jnp.float32),
        pltpu.VMEM((N + L,), jnp.float32),
        pltpu.VMEM((L,), jnp.float32),
        pltpu.SemaphoreType.DMA,
        pltpu.SemaphoreType.DMA,
    ],
)
def _sc_loss(s0_hbm, s1_hbm, ys_hbm, rl_hbm, rr_hbm, part_hbm, *scratch):
    _sc_body(s0_hbm, s1_hbm, ys_hbm, rl_hbm, rr_hbm, part_hbm, *scratch)


def kernel(embeddings, labels):
    n, _ = embeddings.shape
    assert n == N and labels.shape == (N, 2)

    iota = lax.iota(jnp.int32, N)
    sims_l, ys_l, rl_l, rr_l = [], [], [], []
    aux = None
    for d in range(2):
        y = labels[:, d]
        ys, order = lax.sort_key_val(y, iota)
        ys_l.append(ys)
        sims_d, aux_d = _sims(embeddings[order])
        sims_l.append(sims_d)
        if aux is None:
            aux = aux_d
        neq_prev = jnp.concatenate([jnp.ones((1,), jnp.bool_), ys[1:] != ys[:-1]])
        rl_l.append(lax.cummax(jnp.where(neq_prev, iota, 0)))
        neq_next = jnp.concatenate([ys[1:] != ys[:-1], jnp.ones((1,), jnp.bool_)])
        rr_l.append(N - jnp.flip(lax.cummax(jnp.where(jnp.flip(neq_next), iota, 0))))

    offdiag_sims = jnp.sum(aux)
    part = _sc_loss(
        sims_l[0],
        sims_l[1],
        jnp.stack(ys_l),
        jnp.stack(rl_l),
        jnp.stack(rr_l),
    )
    log_sums = jnp.sum(part, axis=(1, 2))
    return (log_sums - offdiag_sims) / (N * (N - 1))
